# Initial kernel scaffold; baseline (speedup 1.0000x reference)
import jax
import jax.numpy as jnp
from jax.experimental import pallas as pl


def kernel(y_pred, y_true, prototypes):
    raise NotImplementedError("write your pallas kernel here")


# speedup vs baseline: 10.2744x; 10.2744x over previous
import functools

import jax
import jax.numpy as jnp
import numpy as np
from jax import lax
from jax.experimental import pallas as pl
from jax.experimental.pallas import tpu as pltpu
from jax.experimental.pallas import tpu_sc as plsc

_TAO = 0.1
_NUM_QUERIES = 16384
_N_CLASS = 21
_EPS = 1e-8

_B, _C, _H, _W = 4, 128, 256, 256
_PLANE = _H * _W

_rng = np.random.RandomState(42)
_IDX = np.sort(_rng.permutation(_B * _PLANE)[:_NUM_QUERIES]).astype(np.int64)
_BQ = _IDX // _PLANE
_PQ = _IDX % _PLANE
_NB = np.bincount(_BQ, minlength=_B)
_NPAD = 4352
assert _NPAD >= int(_NB.max()) and _NPAD % 256 == 0
_QTOT = _B * _NPAD
_PIDX = np.zeros((_B, _NPAD), dtype=np.int32)
_VALID = np.zeros((_B, _NPAD), dtype=bool)
for _b in range(_B):
    _PIDX[_b, : _NB[_b]] = _PQ[_BQ == _b]
    _VALID[_b, : _NB[_b]] = True
_VALID = _VALID.reshape(-1)

_NTILES = 32
_PLANES_PER_TILE = (_B * _C) // _NTILES


def _sc_body(yp_hbm, ytf_hbm, pidx_hbm, feats_hbm, labels_hbm,
             plane_v, pidx_v, out_v):
    wid = lax.axis_index("s") * 2 + lax.axis_index("c")
    b = wid // 8
    c0 = (wid % 8) * _PLANES_PER_TILE

    pltpu.sync_copy(pidx_hbm.at[b], pidx_v)

    def _compact():
        def kbody(k, carry):
            p16 = pidx_v[pl.ds(k * 16, 16)]
            h = jnp.right_shift(p16, 8)
            w = jnp.bitwise_and(p16, 255)
            out_v[pl.ds(k * 16, 16)] = plsc.load_gather(plane_v, [h, w])
            return carry

        lax.fori_loop(0, _NPAD // 16, kbody, 0)

    @pl.when(wid % 8 == 0)
    def _labels():
        pltpu.sync_copy(ytf_hbm.at[b], plane_v)
        _compact()
        pltpu.sync_copy(out_v, labels_hbm.at[pl.ds(b * _NPAD, _NPAD)])

    def jbody(j, carry):
        c = c0 + j
        pltpu.sync_copy(yp_hbm.at[b, c], plane_v)
        _compact()
        pltpu.sync_copy(out_v, feats_hbm.at[c, pl.ds(b * _NPAD, _NPAD)])
        return carry

    lax.fori_loop(0, _PLANES_PER_TILE, jbody, 0)


@functools.cache
def _build_sc_gather():
    mesh = plsc.VectorSubcoreMesh(core_axis_name="c", subcore_axis_name="s")
    return functools.partial(
        pl.kernel,
        mesh=mesh,
        out_type=(
            jax.ShapeDtypeStruct((_C, _QTOT), jnp.float32),
            jax.ShapeDtypeStruct((_QTOT,), jnp.float32),
        ),
        scratch_types=[
            pltpu.VMEM((_H, _W), jnp.float32),
            pltpu.VMEM((_NPAD,), jnp.int32),
            pltpu.VMEM((_NPAD,), jnp.float32),
        ],
        compiler_params=pltpu.CompilerParams(needs_layout_passes=False),
    )(_sc_body)


_NT = 8
_QT = _QTOT // _NT


def _tc_body(feats_ref, labels_ref, protos_ref, out_ref,
             cnt_ref, ssq_ref, loss_ref):
    ph = pl.program_id(0)
    i = pl.program_id(1)
    f = feats_ref[...]
    t = labels_ref[...].reshape(1, _QT)
    nsq = jnp.sum(f * f, axis=0, keepdims=True)
    oh = (lax.broadcasted_iota(jnp.int32, (_N_CLASS, _QT), 0) == t
          ).astype(jnp.float32)

    @pl.when(ph == 0)
    def _phase0():
        @pl.when(i == 0)
        def _init():
            cnt_ref[...] = jnp.zeros_like(cnt_ref)
            ssq_ref[...] = jnp.zeros_like(ssq_ref)
            loss_ref[...] = jnp.zeros_like(loss_ref)

        cnt_ref[...] = cnt_ref[...] + jnp.sum(oh, axis=1, keepdims=True)
        ssq_ref[...] = ssq_ref[...] + jnp.sum(oh * nsq, axis=1, keepdims=True)

    @pl.when(ph == 1)
    def _phase1():
        pr_mat = protos_ref[...]
        count = cnt_ref[...]
        nf = jnp.sqrt(ssq_ref[...])
        base = lax.dot_general(pr_mat, f, (((1,), (0,)), ((), ())),
                               preferred_element_type=jnp.float32)
        na = jnp.sqrt(nsq)
        pn = jnp.sqrt(jnp.sum(pr_mat * pr_mat, axis=1, keepdims=True))
        cos = base / jnp.maximum(na * pn, _EPS)
        s = cos * jnp.abs(cos)
        nf_q = jnp.sum(oh * nf, axis=0, keepdims=True)
        cnt_q = jnp.sum(oh * count, axis=0, keepdims=True)
        z = (nf_q / _TAO) * s
        pos = jnp.mod(t - 1, 20)
        poh = (lax.broadcasted_iota(jnp.int32, (20, _QT), 0) == pos
               ).astype(jnp.float32)
        z0 = jnp.sum(poh * z, axis=0, keepdims=True)
        m = jnp.maximum(jnp.max(z, axis=0, keepdims=True), z0)
        lse = m + jnp.log(jnp.sum(jnp.exp(z - m), axis=0, keepdims=True)
                          + jnp.exp(z0 - m))
        per_row = lse - z0
        contrib = jnp.where(t >= 0, per_row / jnp.maximum(cnt_q, 1.0), 0.0)
        loss_ref[...] = loss_ref[...] + jnp.sum(contrib)

        @pl.when(i == _NT - 1)
        def _final():
            nu = jnp.sum((count > 0.5).astype(jnp.float32))
            out_ref[...] = loss_ref[...] / nu


def _tc_loss(feats, labels, protos):
    return pl.pallas_call(
        _tc_body,
        grid=(2, _NT),
        in_specs=[
            pl.BlockSpec((_C, _QT), lambda p, i: (0, i)),
            pl.BlockSpec((1, 1, _QT), lambda p, i: (i, 0, 0)),
            pl.BlockSpec((20, _C), lambda p, i: (0, 0)),
        ],
        out_specs=pl.BlockSpec((1, 1), lambda p, i: (0, 0)),
        out_shape=jax.ShapeDtypeStruct((1, 1), jnp.float32),
        scratch_shapes=[
            pltpu.VMEM((_N_CLASS, 1), jnp.float32),
            pltpu.VMEM((_N_CLASS, 1), jnp.float32),
            pltpu.VMEM((1, 1), jnp.float32),
        ],
    )(feats, labels, protos)


def kernel(y_pred, y_true, prototypes):
    ytf = lax.bitcast_convert_type(y_true, jnp.float32)
    pidx = jnp.asarray(_PIDX)
    feats, labels_f = _build_sc_gather()(y_pred, ytf, pidx)
    labels_i = lax.bitcast_convert_type(labels_f, jnp.int32)
    labels = jnp.where(jnp.asarray(_VALID), labels_i, jnp.int32(-1))
    out = _tc_loss(feats, labels.reshape(_NT, 1, _QT), prototypes)
    return out[0, 0]


# speedup vs baseline: 12.8919x; 1.2548x over previous
import functools

import jax
import jax.numpy as jnp
import numpy as np
from jax import lax
from jax.experimental import pallas as pl
from jax.experimental.pallas import tpu as pltpu
from jax.experimental.pallas import tpu_sc as plsc

_TAO = 0.1
_NUM_QUERIES = 16384
_N_CLASS = 21
_EPS = 1e-8

_B, _C, _H, _W = 4, 128, 256, 256
_PLANE = _H * _W
_HH = _H // 2
_NSEG = _B * 2

_rng = np.random.RandomState(42)
_IDX = np.sort(_rng.permutation(_B * _PLANE)[:_NUM_QUERIES]).astype(np.int64)
_BQ = _IDX // _PLANE
_PQ = _IDX % _PLANE
_SQ = _BQ * 2 + (_PQ >= _HH * _W).astype(np.int64)
_PLOC = _PQ - (_PQ >= _HH * _W) * (_HH * _W)
_NS = np.bincount(_SQ, minlength=_NSEG)
_NPH = 2176
assert _NPH >= int(_NS.max()) and _NPH % 128 == 0
_QTOT = _NSEG * _NPH
_PIDX = np.zeros((_NSEG, _NPH), dtype=np.int32)
_VALID = np.zeros((_NSEG, _NPH), dtype=bool)
for _s in range(_NSEG):
    _PIDX[_s, : _NS[_s]] = _PLOC[_SQ == _s]
    _VALID[_s, : _NS[_s]] = True
_VALID = _VALID.reshape(-1)

_NTILES = 32
_CPT = _C // 8


def _sc_body(yp_hbm, ytf_hbm, pidx_hbm, feats_hbm, labels_hbm,
             buf_a, buf_b, pidx_v, out_v, sem_a, sem_b):
    wid = lax.axis_index("s") * 2 + lax.axis_index("c")
    b = wid // 8
    c0 = (wid % 8) * _CPT
    bufs = (buf_a, buf_b)
    sems = (sem_a, sem_b)

    pltpu.sync_copy(pidx_hbm.at[pl.ds(b * 2 * _NPH, 2 * _NPH)], pidx_v)

    def _compact(buf, s):
        def kbody(t, carry):
            p16 = pidx_v[pl.ds(s * _NPH + t * 16, 16)]
            h = jnp.right_shift(p16, 8)
            w = jnp.bitwise_and(p16, 255)
            out_v[pl.ds(t * 16, 16)] = plsc.load_gather(buf, [h, w])
            return carry

        lax.fori_loop(0, _NPH // 16, kbody, 0)

    @pl.when(wid % 8 == 0)
    def _labels():
        for s in (0, 1):
            pltpu.sync_copy(ytf_hbm.at[b, pl.ds(s * _HH, _HH)], buf_a)
            _compact(buf_a, s)
            pltpu.sync_copy(
                out_v, labels_hbm.at[pl.ds((b * 2 + s) * _NPH, _NPH)])

    chunks = [(j, s) for j in range(_CPT) for s in (0, 1)]

    def _issue(k):
        j, s = chunks[k]
        return pltpu.async_copy(
            yp_hbm.at[b, c0 + j, pl.ds(s * _HH, _HH)], bufs[k % 2],
            sems[k % 2])

    cp = _issue(0)
    for k in range(len(chunks)):
        nxt = _issue(k + 1) if k + 1 < len(chunks) else None
        cp.wait()
        j, s = chunks[k]
        _compact(bufs[k % 2], s)
        pltpu.sync_copy(
            out_v,
            feats_hbm.at[c0 + j, pl.ds((b * 2 + s) * _NPH, _NPH)])
        cp = nxt


@functools.cache
def _build_sc_gather():
    mesh = plsc.VectorSubcoreMesh(core_axis_name="c", subcore_axis_name="s")
    return functools.partial(
        pl.kernel,
        mesh=mesh,
        out_type=(
            jax.ShapeDtypeStruct((_C, _QTOT), jnp.float32),
            jax.ShapeDtypeStruct((_QTOT,), jnp.float32),
        ),
        scratch_types=[
            pltpu.VMEM((_HH, _W), jnp.float32),
            pltpu.VMEM((_HH, _W), jnp.float32),
            pltpu.VMEM((2 * _NPH,), jnp.int32),
            pltpu.VMEM((_NPH,), jnp.float32),
            pltpu.SemaphoreType.DMA,
            pltpu.SemaphoreType.DMA,
        ],
        compiler_params=pltpu.CompilerParams(needs_layout_passes=False),
    )(_sc_body)


_NT = 8
_QT = _QTOT // _NT


def _tc_body(feats_ref, labels_ref, protos_ref, out_ref,
             cnt_ref, ssq_ref, loss_ref):
    ph = pl.program_id(0)
    i = pl.program_id(1)
    f = feats_ref[...]
    t = labels_ref[...].reshape(1, _QT)
    nsq = jnp.sum(f * f, axis=0, keepdims=True)
    oh = (lax.broadcasted_iota(jnp.int32, (_N_CLASS, _QT), 0) == t
          ).astype(jnp.float32)

    @pl.when(ph == 0)
    def _phase0():
        @pl.when(i == 0)
        def _init():
            cnt_ref[...] = jnp.zeros_like(cnt_ref)
            ssq_ref[...] = jnp.zeros_like(ssq_ref)
            loss_ref[...] = jnp.zeros_like(loss_ref)

        cnt_ref[...] = cnt_ref[...] + jnp.sum(oh, axis=1, keepdims=True)
        ssq_ref[...] = ssq_ref[...] + jnp.sum(oh * nsq, axis=1, keepdims=True)

    @pl.when(ph == 1)
    def _phase1():
        pr_mat = protos_ref[...]
        count = cnt_ref[...]
        nf = jnp.sqrt(ssq_ref[...])
        base = lax.dot_general(pr_mat, f, (((1,), (0,)), ((), ())),
                               preferred_element_type=jnp.float32)
        na = jnp.sqrt(nsq)
        pn = jnp.sqrt(jnp.sum(pr_mat * pr_mat, axis=1, keepdims=True))
        cos = base / jnp.maximum(na * pn, _EPS)
        s = cos * jnp.abs(cos)
        nf_q = jnp.sum(oh * nf, axis=0, keepdims=True)
        cnt_q = jnp.sum(oh * count, axis=0, keepdims=True)
        z = (nf_q / _TAO) * s
        pos = jnp.mod(t - 1, 20)
        poh = (lax.broadcasted_iota(jnp.int32, (20, _QT), 0) == pos
               ).astype(jnp.float32)
        z0 = jnp.sum(poh * z, axis=0, keepdims=True)
        m = jnp.maximum(jnp.max(z, axis=0, keepdims=True), z0)
        lse = m + jnp.log(jnp.sum(jnp.exp(z - m), axis=0, keepdims=True)
                          + jnp.exp(z0 - m))
        per_row = lse - z0
        contrib = jnp.where(t >= 0, per_row / jnp.maximum(cnt_q, 1.0), 0.0)
        loss_ref[...] = loss_ref[...] + jnp.sum(contrib)

        @pl.when(i == _NT - 1)
        def _final():
            nu = jnp.sum((count > 0.5).astype(jnp.float32))
            out_ref[...] = loss_ref[...] / nu


def _tc_loss(feats, labels, protos):
    return pl.pallas_call(
        _tc_body,
        grid=(2, _NT),
        in_specs=[
            pl.BlockSpec((_C, _QT), lambda p, i: (0, i)),
            pl.BlockSpec((1, 1, _QT), lambda p, i: (i, 0, 0)),
            pl.BlockSpec((20, _C), lambda p, i: (0, 0)),
        ],
        out_specs=pl.BlockSpec((1, 1), lambda p, i: (0, 0)),
        out_shape=jax.ShapeDtypeStruct((1, 1), jnp.float32),
        scratch_shapes=[
            pltpu.VMEM((_N_CLASS, 1), jnp.float32),
            pltpu.VMEM((_N_CLASS, 1), jnp.float32),
            pltpu.VMEM((1, 1), jnp.float32),
        ],
    )(feats, labels, protos)


def kernel(y_pred, y_true, prototypes):
    ytf = lax.bitcast_convert_type(y_true, jnp.float32)
    pidx = jnp.asarray(_PIDX.reshape(-1))
    feats, labels_f = _build_sc_gather()(y_pred, ytf, pidx)
    labels_i = lax.bitcast_convert_type(labels_f, jnp.int32)
    labels = jnp.where(jnp.asarray(_VALID), labels_i, jnp.int32(-1))
    out = _tc_loss(feats, labels.reshape(_NT, 1, _QT), prototypes)
    return out[0, 0]
